# baseline (speedup 1.0000x reference)
import functools

import jax
import jax.numpy as jnp
from jax import lax
from jax.experimental import pallas as pl
from jax.experimental.pallas import tpu as pltpu
from jax.experimental.pallas import tpu_sc as plsc

_LANES = 16


def kernel(user_table, item_table, user_ids, item_ids):
    B = user_ids.shape[0]
    D = user_table.shape[1]
    info = plsc.get_sparse_core_info()
    NC, NS = info.num_cores, info.num_subcores
    NW = NC * NS
    bpw = B // NW

    mesh = plsc.VectorSubcoreMesh(core_axis_name="c", subcore_axis_name="s")

    @functools.partial(
        pl.kernel,
        mesh=mesh,
        compiler_params=pltpu.CompilerParams(
            needs_layout_passes=False, use_tc_tiling_on_sc=False),
        out_type=jax.ShapeDtypeStruct((B,), jnp.float32),
        scratch_types=[
            pltpu.VMEM((bpw,), jnp.int32),
            pltpu.VMEM((bpw,), jnp.int32),
            pltpu.VMEM((bpw, D), jnp.float32),
            pltpu.VMEM((bpw, D), jnp.float32),
            pltpu.VMEM((bpw,), jnp.float32),
            pltpu.SemaphoreType.DMA,
            pltpu.SemaphoreType.DMA,
        ],
    )
    def sc_score(ut_hbm, it_hbm, uid_hbm, iid_hbm, out_hbm,
                 uidx_v, iidx_v, u_rows, v_rows, acc_v, sem_u, sem_v):
        wid = lax.axis_index("s") * NC + lax.axis_index("c")
        base = wid * bpw
        pltpu.sync_copy(uid_hbm.at[pl.ds(base, bpw)], uidx_v)
        pltpu.sync_copy(iid_hbm.at[pl.ds(base, bpw)], iidx_v)
        cu = pltpu.async_copy(ut_hbm.at[uidx_v], u_rows, sem_u)
        cv = pltpu.async_copy(it_hbm.at[iidx_v], v_rows, sem_v)
        cu.wait()
        cv.wait()

        def group(g, carry):
            rows = g * _LANES + lax.iota(jnp.int32, _LANES)
            acc = jnp.zeros((_LANES,), jnp.float32)
            for d in range(D):
                cols = jnp.full((_LANES,), d, jnp.int32)
                ug = plsc.load_gather(u_rows, [rows, cols])
                vg = plsc.load_gather(v_rows, [rows, cols])
                acc = acc + ug * vg
            acc_v[pl.ds(g * _LANES, _LANES)] = acc
            return carry

        lax.fori_loop(0, bpw // _LANES, group, 0)
        pltpu.sync_copy(acc_v, out_hbm.at[pl.ds(base, bpw)])

    return sc_score(user_table, item_table, user_ids, item_ids)


# speedup vs baseline: 4.9294x; 4.9294x over previous
import functools

import jax
import jax.numpy as jnp
from jax import lax
from jax.experimental import pallas as pl
from jax.experimental.pallas import tpu as pltpu
from jax.experimental.pallas import tpu_sc as plsc

_LANES = 16
_CH = 2
_GRP = 16


def kernel(user_table, item_table, user_ids, item_ids):
    B = user_ids.shape[0]
    D = user_table.shape[1]
    info = plsc.get_sparse_core_info()
    NC, NS = info.num_cores, info.num_subcores
    NW = NC * NS
    bpw = B // NW

    ut_t = user_table.T
    it_t = item_table.T

    mesh = plsc.VectorSubcoreMesh(core_axis_name="c", subcore_axis_name="s")

    @functools.partial(
        pl.kernel,
        mesh=mesh,
        compiler_params=pltpu.CompilerParams(
            needs_layout_passes=False, use_tc_tiling_on_sc=True),
        out_type=jax.ShapeDtypeStruct((B,), jnp.float32),
        scratch_types=[
            pltpu.VMEM((bpw,), jnp.int32),
            pltpu.VMEM((bpw,), jnp.int32),
            pltpu.VMEM((_CH, D, 128), jnp.float32),
            pltpu.VMEM((_CH, D, 128), jnp.float32),
            pltpu.VMEM((_CH, D, 128), jnp.float32),
            pltpu.VMEM((_CH, D, 128), jnp.float32),
            pltpu.VMEM((_GRP, _LANES), jnp.float32),
            pltpu.VMEM((bpw,), jnp.float32),
            pltpu.SemaphoreType.DMA,
            pltpu.SemaphoreType.DMA,
            pltpu.SemaphoreType.DMA,
            pltpu.SemaphoreType.DMA,
        ],
    )
    def sc_score(ut_hbm, it_hbm, uid_hbm, iid_hbm, out_hbm,
                 uid_v, iid_v,
                 ublk0, ublk1, vblk0, vblk1, pstage, acc_v,
                 sem_u0, sem_u1, sem_v0, sem_v1):
        wid = lax.axis_index("s") * NC + lax.axis_index("c")
        base = wid * bpw
        pltpu.sync_copy(uid_hbm.at[pl.ds(base, bpw)], uid_v)
        pltpu.sync_copy(iid_hbm.at[pl.ds(base, bpw)], iid_v)

        ubufs = (ublk0, ublk1)
        vbufs = (vblk0, vblk1)
        usems = (sem_u0, sem_u1)
        vsems = (sem_v0, sem_v1)
        n_chunks = bpw // _CH

        def fire(c):
            b = c % 2
            g16 = ((c * _CH) // _GRP) * _GRP
            u16 = uid_v[pl.ds(g16, _GRP)]
            i16 = iid_v[pl.ds(g16, _GRP)]
            hs = []
            for j in range(_CH):
                lane = (c * _CH + j) % _GRP
                uc = pl.multiple_of(u16[lane] & -128, 128)
                hs.append(pltpu.async_copy(
                    ut_hbm.at[:, pl.ds(uc, 128)], ubufs[b].at[j], usems[b]))
                ic = pl.multiple_of(i16[lane] & -128, 128)
                hs.append(pltpu.async_copy(
                    it_hbm.at[:, pl.ds(ic, 128)], vbufs[b].at[j], vsems[b]))
            return hs

        handles = fire(0)
        lanes = lax.iota(jnp.int32, _LANES)
        for c in range(n_chunks):
            nxt = fire(c + 1) if c + 1 < n_chunks else []
            for h in handles:
                h.wait()
            handles = nxt
            b = c % 2
            g16 = ((c * _CH) // _GRP) * _GRP
            u16 = uid_v[pl.ds(g16, _GRP)]
            i16 = iid_v[pl.ds(g16, _GRP)]
            for j in range(_CH):
                i = c * _CH + j
                ul = u16[i % _GRP] & 127
                il = i16[i % _GRP] & 127
                p = jnp.zeros((_LANES,), jnp.float32)
                for q in range(D // _LANES):
                    rows = q * _LANES + lanes
                    ug = plsc.load_gather(
                        ubufs[b], [jnp.full((_LANES,), j, jnp.int32), rows,
                                   jnp.full((_LANES,), ul, jnp.int32)])
                    vg = plsc.load_gather(
                        vbufs[b], [jnp.full((_LANES,), j, jnp.int32), rows,
                                   jnp.full((_LANES,), il, jnp.int32)])
                    p = p + ug * vg
                pstage[i % _GRP, :] = p
            if (c * _CH + _CH) % _GRP == 0:
                acc = jnp.zeros((_LANES,), jnp.float32)
                for d in range(_LANES):
                    acc = acc + plsc.load_gather(
                        pstage, [lanes, jnp.full((_LANES,), d, jnp.int32)])
                g = (c * _CH) // _GRP
                acc_v[pl.ds(g * _GRP, _GRP)] = acc

        pltpu.sync_copy(acc_v, out_hbm.at[pl.ds(base, bpw)])

    return sc_score(ut_t, it_t, user_ids, item_ids)


# speedup vs baseline: 5.3339x; 1.0821x over previous
import functools

import jax
import jax.numpy as jnp
from jax import lax
from jax.experimental import pallas as pl
from jax.experimental.pallas import tpu as pltpu
from jax.experimental.pallas import tpu_sc as plsc

_LANES = 16
_CH = 2
_NBUF = 3
_GRP = 16


def kernel(user_table, item_table, user_ids, item_ids):
    B = user_ids.shape[0]
    D = user_table.shape[1]
    info = plsc.get_sparse_core_info()
    NC, NS = info.num_cores, info.num_subcores
    NW = NC * NS
    bpw = B // NW

    ut_t = user_table.T
    it_t = item_table.T

    mesh = plsc.VectorSubcoreMesh(core_axis_name="c", subcore_axis_name="s")

    blk = pltpu.VMEM((_CH, D, 128), jnp.float32)

    @functools.partial(
        pl.kernel,
        mesh=mesh,
        compiler_params=pltpu.CompilerParams(
            needs_layout_passes=False, use_tc_tiling_on_sc=True),
        out_type=jax.ShapeDtypeStruct((B,), jnp.float32),
        scratch_types=[
            pltpu.VMEM((bpw,), jnp.int32),
            pltpu.VMEM((bpw,), jnp.int32),
            blk, blk, blk, blk, blk, blk,
            pltpu.VMEM((_GRP, _LANES), jnp.float32),
            pltpu.VMEM((bpw,), jnp.float32),
        ] + [pltpu.SemaphoreType.DMA] * (2 * _NBUF),
    )
    def sc_score(ut_hbm, it_hbm, uid_hbm, iid_hbm, out_hbm,
                 uid_v, iid_v,
                 ublk0, ublk1, ublk2, vblk0, vblk1, vblk2, pstage, acc_v,
                 sem_u0, sem_u1, sem_u2, sem_v0, sem_v1, sem_v2):
        wid = lax.axis_index("s") * NC + lax.axis_index("c")
        base = wid * bpw
        pltpu.sync_copy(uid_hbm.at[pl.ds(base, bpw)], uid_v)
        pltpu.sync_copy(iid_hbm.at[pl.ds(base, bpw)], iid_v)

        ubufs = (ublk0, ublk1, ublk2)
        vbufs = (vblk0, vblk1, vblk2)
        usems = (sem_u0, sem_u1, sem_u2)
        vsems = (sem_v0, sem_v1, sem_v2)
        n_chunks = bpw // _CH

        def fire(c):
            b = c % _NBUF
            g16 = ((c * _CH) // _GRP) * _GRP
            u16 = uid_v[pl.ds(g16, _GRP)]
            i16 = iid_v[pl.ds(g16, _GRP)]
            hs = []
            for j in range(_CH):
                lane = (c * _CH + j) % _GRP
                uc = pl.multiple_of(u16[lane] & -128, 128)
                hs.append(pltpu.async_copy(
                    ut_hbm.at[:, pl.ds(uc, 128)], ubufs[b].at[j], usems[b]))
                ic = pl.multiple_of(i16[lane] & -128, 128)
                hs.append(pltpu.async_copy(
                    it_hbm.at[:, pl.ds(ic, 128)], vbufs[b].at[j], vsems[b]))
            return hs

        pending = [fire(0)]
        for w in range(1, _NBUF - 1):
            pending.append(fire(w))
        lanes = lax.iota(jnp.int32, _LANES)
        for c in range(n_chunks):
            if c + _NBUF - 1 < n_chunks:
                pending.append(fire(c + _NBUF - 1))
            for h in pending.pop(0):
                h.wait()
            b = c % _NBUF
            g16 = ((c * _CH) // _GRP) * _GRP
            u16 = uid_v[pl.ds(g16, _GRP)]
            i16 = iid_v[pl.ds(g16, _GRP)]
            for j in range(_CH):
                i = c * _CH + j
                ul = u16[i % _GRP] & 127
                il = i16[i % _GRP] & 127
                p = jnp.zeros((_LANES,), jnp.float32)
                for q in range(D // _LANES):
                    rows = q * _LANES + lanes
                    ug = plsc.load_gather(
                        ubufs[b], [jnp.full((_LANES,), j, jnp.int32), rows,
                                   jnp.full((_LANES,), ul, jnp.int32)])
                    vg = plsc.load_gather(
                        vbufs[b], [jnp.full((_LANES,), j, jnp.int32), rows,
                                   jnp.full((_LANES,), il, jnp.int32)])
                    p = p + ug * vg
                pstage[i % _GRP, :] = p
            if (c * _CH + _CH) % _GRP == 0:
                acc = jnp.zeros((_LANES,), jnp.float32)
                for d in range(_LANES):
                    acc = acc + plsc.load_gather(
                        pstage, [lanes, jnp.full((_LANES,), d, jnp.int32)])
                g = (c * _CH) // _GRP
                acc_v[pl.ds(g * _GRP, _GRP)] = acc

        pltpu.sync_copy(acc_v, out_hbm.at[pl.ds(base, bpw)])

    return sc_score(ut_t, it_t, user_ids, item_ids)
